# Initial kernel scaffold; baseline (speedup 1.0000x reference)
import jax
import jax.numpy as jnp
from jax.experimental import pallas as pl


def kernel(indices, weight):
    raise NotImplementedError("write your pallas kernel here")


# speedup vs baseline: 13.7417x; 13.7417x over previous
import jax
import jax.numpy as jnp
from jax.experimental import pallas as pl
from jax.experimental.pallas import tpu as pltpu


def _gather_body(idx_ref, w_ref, o_ref):
    m = o_ref.shape[0]
    for mi in range(m):
        o_ref[mi, 0] = w_ref[idx_ref[0, 0, mi], 0]


def _embedding_gather(flat_idx, weight, *, tokens_per_step=256):
    T = int(flat_idx.shape[0])
    V, D = weight.shape
    m = tokens_per_step

    n_steps = -(-T // m)
    T_pad = n_steps * m
    if T_pad != T:
        flat_idx = jnp.pad(flat_idx, (0, T_pad - T))

    idx3 = flat_idx.reshape(n_steps, 1, m)
    w3 = weight.reshape(V, 1, D)

    table_bytes = V * D * jnp.dtype(weight.dtype).itemsize
    out_block_bytes = m * D * jnp.dtype(weight.dtype).itemsize
    vmem_limit = int(min(table_bytes + 4 * out_block_bytes + (4 << 20),
                         100 * 1024 * 1024))

    out = pl.pallas_call(
        _gather_body,
        out_shape=jax.ShapeDtypeStruct((T_pad, 1, D), weight.dtype),
        grid=(n_steps,),
        in_specs=[
            pl.BlockSpec((1, 1, m), lambda i: (i, 0, 0),
                         memory_space=pltpu.SMEM),
            pl.BlockSpec((V, 1, D), lambda i: (0, 0, 0)),
        ],
        out_specs=pl.BlockSpec((m, 1, D), lambda i: (i, 0, 0)),
        compiler_params=pltpu.CompilerParams(
            dimension_semantics=("parallel",),
            vmem_limit_bytes=vmem_limit,
        ),
    )(idx3, w3)

    return out.reshape(T_pad, D)[:T]


def kernel(indices, weight):
    orig_shape = indices.shape
    flat = indices.reshape(-1).astype(jnp.int32)
    out = _embedding_gather(flat, weight)
    return out.reshape(*orig_shape, weight.shape[1])


# speedup vs baseline: 16.2994x; 1.1861x over previous
import jax
import jax.numpy as jnp
from jax.experimental import pallas as pl
from jax.experimental.pallas import tpu as pltpu


def _gather_body(idx_ref, w_ref, o_ref):
    m = o_ref.shape[0]
    for mi in range(m):
        o_ref[mi, 0] = w_ref[idx_ref[0, 0, mi], 0]


def _embedding_gather(flat_idx, weight, *, tokens_per_step=512):
    T = int(flat_idx.shape[0])
    V, D = weight.shape
    m = tokens_per_step

    n_steps = -(-T // m)
    T_pad = n_steps * m
    if T_pad != T:
        flat_idx = jnp.pad(flat_idx, (0, T_pad - T))

    idx3 = flat_idx.reshape(n_steps, 1, m)
    w3 = weight.reshape(V, 1, D)

    table_bytes = V * D * jnp.dtype(weight.dtype).itemsize
    out_block_bytes = m * D * jnp.dtype(weight.dtype).itemsize
    vmem_limit = int(min(table_bytes + 4 * out_block_bytes + (4 << 20),
                         100 * 1024 * 1024))

    out = pl.pallas_call(
        _gather_body,
        out_shape=jax.ShapeDtypeStruct((T_pad, 1, D), weight.dtype),
        grid=(n_steps,),
        in_specs=[
            pl.BlockSpec((1, 1, m), lambda i: (i, 0, 0),
                         memory_space=pltpu.SMEM),
            pl.BlockSpec((V, 1, D), lambda i: (0, 0, 0)),
        ],
        out_specs=pl.BlockSpec((m, 1, D), lambda i: (i, 0, 0)),
        compiler_params=pltpu.CompilerParams(
            dimension_semantics=("parallel",),
            vmem_limit_bytes=vmem_limit,
        ),
    )(idx3, w3)

    return out.reshape(T_pad, D)[:T]


def kernel(indices, weight):
    orig_shape = indices.shape
    flat = indices.reshape(-1).astype(jnp.int32)
    out = _embedding_gather(flat, weight)
    return out.reshape(*orig_shape, weight.shape[1])


# speedup vs baseline: 17.9344x; 1.1003x over previous
import jax
import jax.numpy as jnp
from jax.experimental import pallas as pl
from jax.experimental.pallas import tpu as pltpu


def _gather_body(idx_ref, w_ref, o_ref):
    m = o_ref.shape[0]
    for mi in range(m):
        o_ref[mi, 0] = w_ref[idx_ref[0, 0, mi], 0]


def _embedding_gather(flat_idx, weight, *, tokens_per_step=1024):
    T = int(flat_idx.shape[0])
    V, D = weight.shape
    m = tokens_per_step

    n_steps = -(-T // m)
    T_pad = n_steps * m
    if T_pad != T:
        flat_idx = jnp.pad(flat_idx, (0, T_pad - T))

    idx3 = flat_idx.reshape(n_steps, 1, m)
    w3 = weight.reshape(V, 1, D)

    table_bytes = V * D * jnp.dtype(weight.dtype).itemsize
    out_block_bytes = m * D * jnp.dtype(weight.dtype).itemsize
    vmem_limit = int(min(table_bytes + 4 * out_block_bytes + (4 << 20),
                         100 * 1024 * 1024))

    out = pl.pallas_call(
        _gather_body,
        out_shape=jax.ShapeDtypeStruct((T_pad, 1, D), weight.dtype),
        grid=(n_steps,),
        in_specs=[
            pl.BlockSpec((1, 1, m), lambda i: (i, 0, 0),
                         memory_space=pltpu.SMEM),
            pl.BlockSpec((V, 1, D), lambda i: (0, 0, 0)),
        ],
        out_specs=pl.BlockSpec((m, 1, D), lambda i: (i, 0, 0)),
        compiler_params=pltpu.CompilerParams(
            dimension_semantics=("parallel",),
            vmem_limit_bytes=vmem_limit,
        ),
    )(idx3, w3)

    return out.reshape(T_pad, D)[:T]


def kernel(indices, weight):
    orig_shape = indices.shape
    flat = indices.reshape(-1).astype(jnp.int32)
    out = _embedding_gather(flat, weight)
    return out.reshape(*orig_shape, weight.shape[1])


# speedup vs baseline: 18.2243x; 1.0162x over previous
import jax
import jax.numpy as jnp
from jax.experimental import pallas as pl
from jax.experimental.pallas import tpu as pltpu


def _gather_body(idx_ref, w_ref, o_ref):
    m = o_ref.shape[0]
    for mi in range(m):
        o_ref[mi, 0] = w_ref[idx_ref[0, 0, mi], 0]


def _embedding_gather(flat_idx, weight, *, tokens_per_step=2048):
    T = int(flat_idx.shape[0])
    V, D = weight.shape
    m = tokens_per_step

    n_steps = -(-T // m)
    T_pad = n_steps * m
    if T_pad != T:
        flat_idx = jnp.pad(flat_idx, (0, T_pad - T))

    idx3 = flat_idx.reshape(n_steps, 1, m)
    w3 = weight.reshape(V, 1, D)

    table_bytes = V * D * jnp.dtype(weight.dtype).itemsize
    out_block_bytes = m * D * jnp.dtype(weight.dtype).itemsize
    vmem_limit = int(min(table_bytes + 4 * out_block_bytes + (4 << 20),
                         100 * 1024 * 1024))

    out = pl.pallas_call(
        _gather_body,
        out_shape=jax.ShapeDtypeStruct((T_pad, 1, D), weight.dtype),
        grid=(n_steps,),
        in_specs=[
            pl.BlockSpec((1, 1, m), lambda i: (i, 0, 0),
                         memory_space=pltpu.SMEM),
            pl.BlockSpec((V, 1, D), lambda i: (0, 0, 0)),
        ],
        out_specs=pl.BlockSpec((m, 1, D), lambda i: (i, 0, 0)),
        compiler_params=pltpu.CompilerParams(
            dimension_semantics=("parallel",),
            vmem_limit_bytes=vmem_limit,
        ),
    )(idx3, w3)

    return out.reshape(T_pad, D)[:T]


def kernel(indices, weight):
    orig_shape = indices.shape
    flat = indices.reshape(-1).astype(jnp.int32)
    out = _embedding_gather(flat, weight)
    return out.reshape(*orig_shape, weight.shape[1])


# speedup vs baseline: 31.4981x; 1.7284x over previous
import jax
import jax.numpy as jnp
from jax.experimental import pallas as pl
from jax.experimental.pallas import tpu as pltpu


def _gather_body(idx_ref, w_ref, o_ref):
    m = o_ref.shape[0]
    for mi in range(m):
        o_ref[mi] = w_ref[idx_ref[0, 0, mi], 0]


def _embedding_gather(flat_idx, weight, *, tokens_per_step=2048):
    T = int(flat_idx.shape[0])
    V, D = weight.shape
    m = tokens_per_step

    n_steps = -(-T // m)
    T_pad = n_steps * m
    if T_pad != T:
        flat_idx = jnp.pad(flat_idx, (0, T_pad - T))

    idx3 = flat_idx.reshape(n_steps, 1, m)
    w3 = weight.reshape(V, 1, D)

    table_bytes = V * D * jnp.dtype(weight.dtype).itemsize
    out_block_bytes = m * D * jnp.dtype(weight.dtype).itemsize
    vmem_limit = int(min(table_bytes + 4 * out_block_bytes + (4 << 20),
                         100 * 1024 * 1024))

    out = pl.pallas_call(
        _gather_body,
        out_shape=jax.ShapeDtypeStruct((T_pad, D), weight.dtype),
        grid=(n_steps,),
        in_specs=[
            pl.BlockSpec((1, 1, m), lambda i: (i, 0, 0),
                         memory_space=pltpu.SMEM),
            pl.BlockSpec((V, 1, D), lambda i: (0, 0, 0)),
        ],
        out_specs=pl.BlockSpec((m, D), lambda i: (i, 0)),
        compiler_params=pltpu.CompilerParams(
            dimension_semantics=("parallel",),
            vmem_limit_bytes=vmem_limit,
        ),
    )(idx3, w3)

    return out[:T]


def kernel(indices, weight):
    orig_shape = indices.shape
    flat = indices.reshape(-1).astype(jnp.int32)
    out = _embedding_gather(flat, weight)
    return out.reshape(*orig_shape, weight.shape[1])
